# Initial kernel scaffold; baseline (speedup 1.0000x reference)
import jax
import jax.numpy as jnp
from jax.experimental import pallas as pl


def kernel(x, embed_table, W, b):
    raise NotImplementedError("write your pallas kernel here")


# speedup vs baseline: 2.4506x; 2.4506x over previous
import functools

import jax
import jax.numpy as jnp
from jax import lax
from jax.experimental import pallas as pl
from jax.experimental.pallas import tpu as pltpu
from jax.experimental.pallas import tpu_sc as plsc

VOCAB = 1000000
EMBED_DIM = 64
SEQ = 50
BATCH = 16384
LANES = 16


def _pooling_sc(xt_flat, embed_table):
    info = plsc.get_sparse_core_info()
    nc, ns = info.num_cores, info.num_subcores
    nw = nc * ns
    b_per_w = BATCH // nw
    cb = 32
    n_chunks = b_per_w // cb
    rows_per_chunk = cb * SEQ

    mesh = plsc.VectorSubcoreMesh(core_axis_name="c", subcore_axis_name="s")

    @functools.partial(
        pl.kernel,
        mesh=mesh,
        compiler_params=pltpu.CompilerParams(use_tc_tiling_on_sc=False),
        out_type=jax.ShapeDtypeStruct((BATCH, EMBED_DIM), jnp.float32),
        scratch_types=[
            pltpu.VMEM((rows_per_chunk,), jnp.int32),
            pltpu.VMEM((rows_per_chunk, EMBED_DIM), jnp.float32),
            pltpu.VMEM((cb, EMBED_DIM), jnp.float32),
            pltpu.SemaphoreType.DMA,
        ],
    )
    def sc_kernel(xt_hbm, table_hbm, out_hbm, idx_v, rows_v, acc_v, sem):
        wid = lax.axis_index("s") * nc + lax.axis_index("c")
        for c in range(n_chunks):
            elem_base = (wid * b_per_w + c * cb) * SEQ
            pltpu.sync_copy(xt_hbm.at[pl.ds(elem_base, rows_per_chunk)], idx_v)
            pltpu.async_copy(table_hbm.at[idx_v], rows_v, sem).wait()

            def b_body(bi, _):
                def j_body(j, accs):
                    r = bi * SEQ + j
                    return tuple(
                        accs[q] + rows_v[r, pl.ds(q * LANES, LANES)]
                        for q in range(EMBED_DIM // LANES)
                    )

                accs = lax.fori_loop(
                    0, SEQ, j_body,
                    tuple(jnp.zeros((LANES,), jnp.float32)
                          for _ in range(EMBED_DIM // LANES)),
                )
                for q in range(EMBED_DIM // LANES):
                    acc_v[bi, pl.ds(q * LANES, LANES)] = accs[q]
                return 0

            lax.fori_loop(0, cb, b_body, 0)
            out_base = wid * b_per_w + c * cb
            pltpu.sync_copy(acc_v, out_hbm.at[pl.ds(out_base, cb), :])

    return sc_kernel(xt_flat, embed_table)


def _tail_tc(sums, W, b2):
    blk = 2048

    def tail_kernel(s_ref, w_ref, b_ref, o_ref):
        s = s_ref[...]
        w = w_ref[...]
        bb = b_ref[...]
        z0 = jnp.sum(s * w[0:1, :], axis=1, keepdims=True) + bb[:, 0:1]
        z1 = jnp.sum(s * w[1:2, :], axis=1, keepdims=True) + bb[:, 1:2]
        s0 = jax.nn.sigmoid(z0)
        s1 = jax.nn.sigmoid(z1)
        m = jnp.maximum(s0, s1)
        lse = m + jnp.log(jnp.exp(s0 - m) + jnp.exp(s1 - m))
        o_ref[...] = jnp.concatenate([s0 - lse, s1 - lse], axis=1)

    return pl.pallas_call(
        tail_kernel,
        grid=(BATCH // blk,),
        in_specs=[
            pl.BlockSpec((blk, EMBED_DIM), lambda i: (i, 0)),
            pl.BlockSpec((2, EMBED_DIM), lambda i: (0, 0)),
            pl.BlockSpec((1, 2), lambda i: (0, 0)),
        ],
        out_specs=pl.BlockSpec((blk, 2), lambda i: (i, 0)),
        out_shape=jax.ShapeDtypeStruct((BATCH, 2), jnp.float32),
    )(sums, W, b2)


def kernel(x, embed_table, W, b):
    xt_flat = x.astype(jnp.int32).T.reshape(-1)
    sums = _pooling_sc(xt_flat, embed_table)
    return _tail_tc(sums, W, b.reshape(1, 2))


# speedup vs baseline: 2.8129x; 1.1478x over previous
import functools

import jax
import jax.numpy as jnp
from jax import lax
from jax.experimental import pallas as pl
from jax.experimental.pallas import tpu as pltpu
from jax.experimental.pallas import tpu_sc as plsc

VOCAB = 1000000
EMBED_DIM = 64
SEQ = 50
BATCH = 16384
LANES = 16


def _pooling_sc(x2d, embed_table):
    info = plsc.get_sparse_core_info()
    nc, ns = info.num_cores, info.num_subcores
    nw = nc * ns
    b_per_w = BATCH // nw

    mesh = plsc.VectorSubcoreMesh(core_axis_name="c", subcore_axis_name="s")

    @functools.partial(
        pl.kernel,
        mesh=mesh,
        compiler_params=pltpu.CompilerParams(use_tc_tiling_on_sc=False),
        out_type=jax.ShapeDtypeStruct((BATCH, EMBED_DIM), jnp.float32),
        scratch_types=[
            pltpu.VMEM((SEQ, b_per_w), jnp.int32),
            pltpu.VMEM((b_per_w, EMBED_DIM), jnp.float32),
            pltpu.SemaphoreType.DMA,
        ],
    )
    def sc_kernel(x_hbm, table_hbm, out_hbm, idx_v, acc_v, sem):
        wid = lax.axis_index("s") * nc + lax.axis_index("c")
        base = wid * b_per_w
        pltpu.sync_copy(x_hbm.at[:, pl.ds(base, b_per_w)], idx_v)

        zf = jnp.zeros((LANES,), jnp.float32)

        def z_body(i, _):
            for q in range(EMBED_DIM // LANES):
                acc_v[i, pl.ds(q * LANES, LANES)] = zf
            return 0

        lax.fori_loop(0, b_per_w, z_body, 0)

        copies = [
            pltpu.async_copy(table_hbm.at[idx_v.at[j]], acc_v, sem, add=True)
            for j in range(SEQ)
        ]
        for cp in copies:
            cp.wait()
        pltpu.sync_copy(acc_v, out_hbm.at[pl.ds(base, b_per_w), :])

    return sc_kernel(x2d, embed_table)


def _tail_tc(sums, W, b2):
    blk = 2048

    def tail_kernel(s_ref, w_ref, b_ref, o_ref):
        s = s_ref[...]
        w = w_ref[...]
        bb = b_ref[...]
        z0 = jnp.sum(s * w[0:1, :], axis=1, keepdims=True) + bb[:, 0:1]
        z1 = jnp.sum(s * w[1:2, :], axis=1, keepdims=True) + bb[:, 1:2]
        s0 = jax.nn.sigmoid(z0)
        s1 = jax.nn.sigmoid(z1)
        m = jnp.maximum(s0, s1)
        lse = m + jnp.log(jnp.exp(s0 - m) + jnp.exp(s1 - m))
        o_ref[...] = jnp.concatenate([s0 - lse, s1 - lse], axis=1)

    return pl.pallas_call(
        tail_kernel,
        grid=(BATCH // blk,),
        in_specs=[
            pl.BlockSpec((blk, EMBED_DIM), lambda i: (i, 0)),
            pl.BlockSpec((2, EMBED_DIM), lambda i: (0, 0)),
            pl.BlockSpec((1, 2), lambda i: (0, 0)),
        ],
        out_specs=pl.BlockSpec((blk, 2), lambda i: (i, 0)),
        out_shape=jax.ShapeDtypeStruct((BATCH, 2), jnp.float32),
    )(sums, W, b2)


def kernel(x, embed_table, W, b):
    sums = _pooling_sc(x.astype(jnp.int32), embed_table)
    return _tail_tc(sums, W, b.reshape(1, 2))
